# baseline (speedup 1.0000x reference)
import functools

import jax
import jax.numpy as jnp
from jax import lax
from jax.experimental import pallas as pl
from jax.experimental.pallas import tpu as pltpu
from jax.experimental.pallas import tpu_sc as plsc

NC = 2
NS = 16
LANES = 16
NBUF = 2


def _noisy_or(rules_flat, g2l, w_flat, *, B, L, num_rel):
    NW = NC * NS
    n_g2l = g2l.shape[0]
    n_ids = n_g2l - 1
    chunk_unit = NS * LANES
    T_pad = ((n_ids + chunk_unit - 1) // chunk_unit) * chunk_unit
    chunk = T_pad // NS
    tail = n_ids - (NS - 1) * chunk
    assert 0 < tail <= chunk and tail % 8 == 0
    w_copy = (num_rel + 1) // 8 * 8
    W_pad = ((num_rel + 1 + chunk_unit - 1) // chunk_unit) * chunk_unit
    f_chunk = W_pad // NS
    rows_w = B // NW
    groups = rows_w // LANES
    assert groups % NBUF == 0
    gl = LANES * L

    mesh = plsc.VectorSubcoreMesh(core_axis_name="c", subcore_axis_name="s")

    @functools.partial(
        pl.kernel,
        out_type=jax.ShapeDtypeStruct((B,), jnp.float32),
        mesh=mesh,
        compiler_params=pltpu.CompilerParams(needs_layout_passes=False,
                                             use_tc_tiling_on_sc=True),
        scratch_types=[
            pltpu.VMEM((W_pad,), jnp.float32),
            pltpu.VMEM((chunk,), jnp.int32),
            pltpu.VMEM((T_pad,), jnp.float32),
            pltpu.VMEM_SHARED((T_pad,), jnp.float32),
            [pltpu.VMEM((LANES, L), jnp.int32) for _ in range(NBUF)],
            pltpu.VMEM((rows_w,), jnp.float32),
            [pltpu.SemaphoreType.DMA for _ in range(NBUF)],
        ],
    )
    def run(rules_hbm, g2l_hbm, w_hbm, out_hbm,
            w_v, g2l_v, t_v, t_sh, rbufs, o_v, sems):
        cid = lax.axis_index("c")
        sid = lax.axis_index("s")
        wid = sid * NC + cid
        lane = lax.iota(jnp.int32, LANES)
        row_base = wid * rows_w

        def rules_src(g):
            return rules_hbm.at[pl.ds(row_base + g * LANES, LANES), :]

        for b in range(NBUF):
            pltpu.async_copy(rules_src(b), rbufs[b], sems[b])

        with jax.named_scope("build"):
            pltpu.sync_copy(w_hbm.at[pl.ds(0, w_copy)],
                            w_v.at[pl.ds(0, w_copy)])

            @pl.when(sid < NS - 1)
            def _():
                pltpu.sync_copy(g2l_hbm.at[pl.ds(sid * chunk, chunk)], g2l_v)

            @pl.when(sid == NS - 1)
            def _():
                pltpu.sync_copy(g2l_hbm.at[pl.ds((NS - 1) * chunk, tail)],
                                g2l_v.at[pl.ds(0, tail)])

            f_base = sid * f_chunk

            @plsc.parallel_loop(0, f_chunk // LANES, unroll=2)
            def _(i):
                w = w_v[pl.ds(f_base + i * LANES, LANES)]
                f = 1.0 / (1.0 + jnp.exp(w))
                f = jnp.where(f_base + i * LANES + lane >= num_rel, 1.0, f)
                w_v[pl.ds(f_base + i * LANES, LANES)] = f

            pltpu.sync_copy(w_v.at[pl.ds(f_base, f_chunk)],
                            t_sh.at[pl.ds(f_base, f_chunk)])
            plsc.subcore_barrier()
            pltpu.sync_copy(t_sh.at[pl.ds(0, W_pad)], w_v)
            plsc.subcore_barrier()

            limit = jnp.where(sid == NS - 1, tail, chunk)

            @plsc.parallel_loop(0, chunk // LANES, unroll=2)
            def _(i):
                idx = g2l_v[pl.ds(i * LANES, LANES)]
                idx = jnp.where(i * LANES + lane < limit, idx, num_rel)
                t_v[pl.ds(i * LANES, LANES)] = plsc.load_gather(w_v, [idx])

        with jax.named_scope("bcast"):
            pltpu.sync_copy(t_v.at[pl.ds(0, chunk)],
                            t_sh.at[pl.ds(sid * chunk, chunk)])
            plsc.subcore_barrier()
            pltpu.sync_copy(t_sh, t_v)

        ones = jnp.ones((LANES,), jnp.float32)

        def body(j, carry):
            for sub in range(NBUF):
                g = j * NBUF + sub
                rbuf, sem = rbufs[sub], sems[sub]
                pltpu.make_async_copy(rules_src(0), rbuf, sem).wait()

                @plsc.parallel_loop(0, L // 4, unroll=2,
                                    carry=(ones, ones, ones, ones))
                def accs(i, c):
                    a0, a1, a2, a3 = c
                    l = jnp.full((LANES,), i * 4, jnp.int32)
                    i0 = plsc.load_gather(rbuf, [lane, l])
                    i1 = plsc.load_gather(rbuf, [lane, l + 1])
                    i2 = plsc.load_gather(rbuf, [lane, l + 2])
                    i3 = plsc.load_gather(rbuf, [lane, l + 3])
                    f0 = plsc.load_gather(t_v, [i0])
                    f1 = plsc.load_gather(t_v, [i1])
                    f2 = plsc.load_gather(t_v, [i2])
                    f3 = plsc.load_gather(t_v, [i3])
                    return (a0 * f0, a1 * f1, a2 * f2, a3 * f3)

                @pl.when(g + NBUF < groups)
                def _():
                    pltpu.async_copy(rules_src(g + NBUF), rbuf, sem)

                a0, a1, a2, a3 = accs
                prod = (a0 * a1) * (a2 * a3)
                res = jnp.clip(1.0 - prod, 0.0001, 0.99999)
                o_v[pl.ds(g * LANES, LANES)] = res
            return carry

        with jax.named_scope("main"):
            lax.fori_loop(0, groups // NBUF, body, 0)
            pltpu.sync_copy(o_v, out_hbm.at[pl.ds(row_base, rows_w)])

    return run(rules_flat, g2l, w_flat)


def kernel(rules, global_to_local, weights):
    B, L = rules.shape
    num_rel = weights.shape[0] - 1
    out = _noisy_or(rules, global_to_local,
                    weights.reshape(-1), B=B, L=L, num_rel=num_rel)
    return out.reshape(B, 1)


# speedup vs baseline: 1.4603x; 1.4603x over previous
import functools

import jax
import jax.numpy as jnp
from jax import lax
from jax.experimental import pallas as pl
from jax.experimental.pallas import tpu as pltpu
from jax.experimental.pallas import tpu_sc as plsc

NC = 2
NS = 16
LANES = 16
NBUF = 2


def _noisy_or(rules_flat, g2l, w_flat, *, B, L, num_rel):
    NW = NC * NS
    n_g2l = g2l.shape[0]
    n_ids = n_g2l - 1
    chunk_unit = NS * LANES
    T_pad = ((n_ids + chunk_unit - 1) // chunk_unit) * chunk_unit
    chunk = T_pad // NS
    tail = n_ids - (NS - 1) * chunk
    assert 0 < tail <= chunk and tail % 8 == 0
    w_copy = (num_rel + 1) // 8 * 8
    W_pad = ((num_rel + 1 + chunk_unit - 1) // chunk_unit) * chunk_unit
    f_chunk = W_pad // NS
    rows_w = B // NW
    groups = rows_w // LANES
    assert groups % NBUF == 0
    gl = LANES * L

    mesh = plsc.VectorSubcoreMesh(core_axis_name="c", subcore_axis_name="s")

    @functools.partial(
        pl.kernel,
        out_type=jax.ShapeDtypeStruct((B,), jnp.float32),
        mesh=mesh,
        compiler_params=pltpu.CompilerParams(needs_layout_passes=False,
                                             use_tc_tiling_on_sc=True),
        scratch_types=[
            pltpu.VMEM((W_pad,), jnp.float32),
            pltpu.VMEM((chunk,), jnp.int32),
            pltpu.VMEM((T_pad,), jnp.float32),
            pltpu.VMEM_SHARED((T_pad,), jnp.float32),
            [pltpu.VMEM((LANES, L), jnp.int32) for _ in range(NBUF)],
            pltpu.VMEM((LANES * LANES,), jnp.float32),
            pltpu.VMEM((rows_w,), jnp.float32),
            [pltpu.SemaphoreType.DMA for _ in range(NBUF)],
        ],
    )
    def run(rules_hbm, g2l_hbm, w_hbm, out_hbm,
            w_v, g2l_v, t_v, t_sh, rbufs, m_v, o_v, sems):
        cid = lax.axis_index("c")
        sid = lax.axis_index("s")
        wid = sid * NC + cid
        lane = lax.iota(jnp.int32, LANES)
        row_base = wid * rows_w

        def rules_src(g):
            return rules_hbm.at[pl.ds(row_base + g * LANES, LANES), :]

        for b in range(NBUF):
            pltpu.async_copy(rules_src(b), rbufs[b], sems[b])

        with jax.named_scope("build"):
            pltpu.sync_copy(w_hbm.at[pl.ds(0, w_copy)],
                            w_v.at[pl.ds(0, w_copy)])

            @pl.when(sid < NS - 1)
            def _():
                pltpu.sync_copy(g2l_hbm.at[pl.ds(sid * chunk, chunk)], g2l_v)

            @pl.when(sid == NS - 1)
            def _():
                pltpu.sync_copy(g2l_hbm.at[pl.ds((NS - 1) * chunk, tail)],
                                g2l_v.at[pl.ds(0, tail)])

            f_base = sid * f_chunk

            @plsc.parallel_loop(0, f_chunk // LANES, unroll=2)
            def _(i):
                w = w_v[pl.ds(f_base + i * LANES, LANES)]
                f = 1.0 / (1.0 + jnp.exp(w))
                f = jnp.where(f_base + i * LANES + lane >= num_rel, 1.0, f)
                w_v[pl.ds(f_base + i * LANES, LANES)] = f

            pltpu.sync_copy(w_v.at[pl.ds(f_base, f_chunk)],
                            t_sh.at[pl.ds(f_base, f_chunk)])
            plsc.subcore_barrier()
            pltpu.sync_copy(t_sh.at[pl.ds(0, W_pad)], w_v)
            plsc.subcore_barrier()

            limit = jnp.where(sid == NS - 1, tail, chunk)

            @plsc.parallel_loop(0, chunk // LANES, unroll=2)
            def _(i):
                idx = g2l_v[pl.ds(i * LANES, LANES)]
                idx = jnp.where(i * LANES + lane < limit, idx, num_rel)
                t_v[pl.ds(i * LANES, LANES)] = plsc.load_gather(w_v, [idx])

        with jax.named_scope("bcast"):
            pltpu.sync_copy(t_v.at[pl.ds(0, chunk)],
                            t_sh.at[pl.ds(sid * chunk, chunk)])
            plsc.subcore_barrier()
            pltpu.sync_copy(t_sh, t_v)

        ones = jnp.ones((LANES,), jnp.float32)
        n_full = L // LANES
        offs = [k * LANES for k in range(n_full)]
        ragged = L % LANES != 0
        if ragged:
            offs.append(L - LANES)
        lane16 = lane * LANES

        def body(j, carry):
            for sub in range(NBUF):
                g = j * NBUF + sub
                rbuf, sem = rbufs[sub], sems[sub]
                pltpu.make_async_copy(rules_src(0), rbuf, sem).wait()

                for r in range(LANES):
                    a0, a1 = ones, ones
                    for k, off in enumerate(offs):
                        ids = rbuf[r, pl.ds(off, LANES)]
                        f = plsc.load_gather(t_v, [ids])
                        if ragged and k == len(offs) - 1:
                            dup = (n_full + 1) * LANES - L
                            f = jnp.where(lane < dup, 1.0, f)
                        if k % 2 == 0:
                            a0 = a0 * f
                        else:
                            a1 = a1 * f
                    m_v[pl.ds(r * LANES, LANES)] = a0 * a1

                @pl.when(g + NBUF < groups)
                def _():
                    pltpu.async_copy(rules_src(g + NBUF), rbuf, sem)

                p0, p1, p2, p3 = ones, ones, ones, ones
                for k in range(0, LANES, 4):
                    p0 = p0 * plsc.load_gather(m_v, [lane16 + k])
                    p1 = p1 * plsc.load_gather(m_v, [lane16 + (k + 1)])
                    p2 = p2 * plsc.load_gather(m_v, [lane16 + (k + 2)])
                    p3 = p3 * plsc.load_gather(m_v, [lane16 + (k + 3)])
                prod = (p0 * p1) * (p2 * p3)
                res = jnp.clip(1.0 - prod, 0.0001, 0.99999)
                o_v[pl.ds(g * LANES, LANES)] = res
            return carry

        with jax.named_scope("main"):
            lax.fori_loop(0, groups // NBUF, body, 0)
            pltpu.sync_copy(o_v, out_hbm.at[pl.ds(row_base, rows_w)])

    return run(rules_flat, g2l, w_flat)


def kernel(rules, global_to_local, weights):
    B, L = rules.shape
    num_rel = weights.shape[0] - 1
    out = _noisy_or(rules, global_to_local,
                    weights.reshape(-1), B=B, L=L, num_rel=num_rel)
    return out.reshape(B, 1)


# speedup vs baseline: 1.5092x; 1.0335x over previous
import functools

import jax
import jax.numpy as jnp
from jax import lax
from jax.experimental import pallas as pl
from jax.experimental.pallas import tpu as pltpu
from jax.experimental.pallas import tpu_sc as plsc

NC = 2
NS = 16
LANES = 16
NBUF = 3


def _noisy_or(rules_flat, g2l, w_flat, *, B, L, num_rel):
    NW = NC * NS
    n_g2l = g2l.shape[0]
    n_ids = n_g2l - 1
    chunk_unit = NS * LANES
    T_pad = ((n_ids + chunk_unit - 1) // chunk_unit) * chunk_unit
    chunk = T_pad // NS
    tail = n_ids - (NS - 1) * chunk
    assert 0 < tail <= chunk and tail % 8 == 0
    half = (chunk // 2 + LANES - 1) // LANES * LANES
    halves = [(0, half), (half, chunk - half)]
    assert all(sz % 8 == 0 and sz > 0 for _, sz in halves)
    w_copy = (num_rel + 1) // 8 * 8
    W_pad = ((num_rel + 1 + chunk_unit - 1) // chunk_unit) * chunk_unit
    f_chunk = W_pad // NS
    rows_w = B // NW
    groups = rows_w // LANES
    gl = LANES * L

    mesh = plsc.VectorSubcoreMesh(core_axis_name="c", subcore_axis_name="s")

    @functools.partial(
        pl.kernel,
        out_type=jax.ShapeDtypeStruct((B,), jnp.float32),
        mesh=mesh,
        compiler_params=pltpu.CompilerParams(needs_layout_passes=False,
                                             use_tc_tiling_on_sc=True),
        scratch_types=[
            pltpu.VMEM((W_pad,), jnp.float32),
            pltpu.VMEM((half,), jnp.int32),
            pltpu.VMEM((T_pad,), jnp.float32),
            pltpu.VMEM_SHARED((T_pad,), jnp.float32),
            [pltpu.VMEM((LANES, L), jnp.int32) for _ in range(NBUF)],
            pltpu.VMEM((LANES * LANES,), jnp.float32),
            pltpu.VMEM((rows_w,), jnp.float32),
            [pltpu.SemaphoreType.DMA for _ in range(NBUF)],
        ],
    )
    def run(rules_hbm, g2l_hbm, w_hbm, out_hbm,
            w_v, g2l_v, t_v, t_sh, rbufs, m_v, o_v, sems):
        cid = lax.axis_index("c")
        sid = lax.axis_index("s")
        wid = sid * NC + cid
        lane = lax.iota(jnp.int32, LANES)
        row_base = wid * rows_w

        def rules_src(g):
            return rules_hbm.at[pl.ds(row_base + g * LANES, LANES), :]

        with jax.named_scope("build"):
            pltpu.sync_copy(w_hbm.at[pl.ds(0, w_copy)],
                            w_v.at[pl.ds(0, w_copy)])

            def stage_g2l(h_off, h_sz):
                part = min(h_sz, max(0, tail - h_off))
                assert part % 8 == 0

                @pl.when(sid < NS - 1)
                def _():
                    pltpu.sync_copy(
                        g2l_hbm.at[pl.ds(sid * chunk + h_off, h_sz)],
                        g2l_v.at[pl.ds(0, h_sz)])

                if part > 0:
                    @pl.when(sid == NS - 1)
                    def _():
                        pltpu.sync_copy(
                            g2l_hbm.at[pl.ds((NS - 1) * chunk + h_off, part)],
                            g2l_v.at[pl.ds(0, part)])

            for b in range(NBUF):
                pltpu.async_copy(rules_src(b), rbufs[b], sems[b])

            f_base = sid * f_chunk

            @plsc.parallel_loop(0, f_chunk // LANES, unroll=2)
            def _(i):
                w = w_v[pl.ds(f_base + i * LANES, LANES)]
                f = 1.0 / (1.0 + jnp.exp(w))
                f = jnp.where(f_base + i * LANES + lane >= num_rel, 1.0, f)
                w_v[pl.ds(f_base + i * LANES, LANES)] = f

            pltpu.sync_copy(w_v.at[pl.ds(f_base, f_chunk)],
                            t_sh.at[pl.ds(f_base, f_chunk)])
            plsc.subcore_barrier()
            pltpu.sync_copy(t_sh.at[pl.ds(0, W_pad)], w_v)
            plsc.subcore_barrier()

            limit = jnp.where(sid == NS - 1, tail, chunk)
            for h_off, h_sz in halves:
                stage_g2l(h_off, h_sz)
                lim_h = limit - h_off

                @plsc.parallel_loop(0, h_sz // LANES, unroll=4)
                def _(i):
                    idx = g2l_v[pl.ds(i * LANES, LANES)]
                    idx = jnp.where(i * LANES + lane < lim_h, idx, num_rel)
                    t_v[pl.ds(h_off + i * LANES, LANES)] = (
                        plsc.load_gather(w_v, [idx]))

        with jax.named_scope("bcast"):
            pltpu.sync_copy(t_v.at[pl.ds(0, chunk)],
                            t_sh.at[pl.ds(sid * chunk, chunk)])
            plsc.subcore_barrier()
            pltpu.sync_copy(t_sh, t_v)

        ones = jnp.ones((LANES,), jnp.float32)
        n_full = L // LANES
        offs = [k * LANES for k in range(n_full)]
        ragged = L % LANES != 0
        if ragged:
            offs.append(L - LANES)
        lane16 = lane * LANES

        def compute_group(g, sub, prefetch):
            rbuf, sem = rbufs[sub], sems[sub]
            pltpu.make_async_copy(rules_src(0), rbuf, sem).wait()

            for r in range(LANES):
                a0, a1 = ones, ones
                for k, off in enumerate(offs):
                    ids = rbuf[r, pl.ds(off, LANES)]
                    f = plsc.load_gather(t_v, [ids])
                    if ragged and k == len(offs) - 1:
                        dup = (n_full + 1) * LANES - L
                        f = jnp.where(lane < dup, 1.0, f)
                    if k % 2 == 0:
                        a0 = a0 * f
                    else:
                        a1 = a1 * f
                m_v[pl.ds(r * LANES, LANES)] = a0 * a1

            if prefetch:
                @pl.when(g + NBUF < groups)
                def _():
                    pltpu.async_copy(rules_src(g + NBUF), rbuf, sem)

            p0, p1, p2, p3 = ones, ones, ones, ones
            for k in range(0, LANES, 4):
                p0 = p0 * plsc.load_gather(m_v, [lane16 + k])
                p1 = p1 * plsc.load_gather(m_v, [lane16 + (k + 1)])
                p2 = p2 * plsc.load_gather(m_v, [lane16 + (k + 2)])
                p3 = p3 * plsc.load_gather(m_v, [lane16 + (k + 3)])
            prod = (p0 * p1) * (p2 * p3)
            res = jnp.clip(1.0 - prod, 0.0001, 0.99999)
            o_v[pl.ds(g * LANES, LANES)] = res

        def body(j, carry):
            for sub in range(NBUF):
                compute_group(j * NBUF + sub, sub, True)
            return carry

        with jax.named_scope("main"):
            loop_groups = groups // NBUF * NBUF
            lax.fori_loop(0, groups // NBUF, body, 0)
            for g in range(loop_groups, groups):
                compute_group(g, g % NBUF, False)
            pltpu.sync_copy(o_v, out_hbm.at[pl.ds(row_base, rows_w)])

    return run(rules_flat, g2l, w_flat)


def kernel(rules, global_to_local, weights):
    B, L = rules.shape
    num_rel = weights.shape[0] - 1
    out = _noisy_or(rules, global_to_local,
                    weights.reshape(-1), B=B, L=L, num_rel=num_rel)
    return out.reshape(B, 1)
